# baseline (speedup 1.0000x reference)
import functools

import jax
import jax.numpy as jnp
from jax import lax
from jax.experimental import pallas as pl
from jax.experimental.pallas import tpu as pltpu
from jax.experimental.pallas import tpu_sc as plsc

N = 10000
E = 320000
F_IN = 128
H = 64
H2 = 32
C = 40

_BN_INV = (1.0 + 1e-5) ** -0.5

NW = 32
EPT = E // NW
G = 80
CH = 2000
NG = CH // G
NCHUNK = EPT // CH
RPT = N // 16
RB = 2000
GRID = N // RB


def _tc_fwd_body(x_ref, w1_ref, b1_ref, wc_ref, g1_ref, bb1_ref, h_ref, hw_ref):
    h = jnp.dot(x_ref[...], w1_ref[...], preferred_element_type=jnp.float32)
    h = h + b1_ref[...][None, :]
    h = jnp.where(h >= 0, h, 0.01 * h)
    h = h * (g1_ref[...] * _BN_INV)[None, :] + bb1_ref[...][None, :]
    h_ref[...] = h
    hw_ref[...] = jnp.dot(h, wc_ref[...], preferred_element_type=jnp.float32)


def _tc_fwd(x, W1, b1, Wc, bn1_g, bn1_b):
    return pl.pallas_call(
        _tc_fwd_body,
        grid=(GRID,),
        in_specs=[
            pl.BlockSpec((RB, F_IN), lambda i: (i, 0)),
            pl.BlockSpec((F_IN, H), lambda i: (0, 0)),
            pl.BlockSpec((H,), lambda i: (0,)),
            pl.BlockSpec((H, H2), lambda i: (0, 0)),
            pl.BlockSpec((H,), lambda i: (0,)),
            pl.BlockSpec((H,), lambda i: (0,)),
        ],
        out_specs=[
            pl.BlockSpec((RB, H), lambda i: (i, 0)),
            pl.BlockSpec((RB, H2), lambda i: (i, 0)),
        ],
        out_shape=[
            jax.ShapeDtypeStruct((N, H), jnp.float32),
            jax.ShapeDtypeStruct((N, H2), jnp.float32),
        ],
    )(x, W1, b1, Wc, bn1_g, bn1_b)


NR = N // 16
TRW = 40


def _sc_deg_body(ei, ew, out, deg_sh, src_v, cidx_f, ew_f, dstage_v, pk_v,
                 sem_sc):
    cid = lax.axis_index("c")
    sid = lax.axis_index("s")
    wid = sid * 2 + cid

    z = jnp.zeros((16,), jnp.float32)
    z_i = jnp.zeros((16,), jnp.int32)
    iota = lax.iota(jnp.int32, 16)

    def zloop(i, _):
        src_v[i, pl.ds(0, 16)] = z
        return 0

    lax.fori_loop(0, CH, zloop, 0)
    pltpu.sync_copy(src_v.at[pl.ds(0, RPT)],
                    deg_sh.at[pl.ds(sid * RPT, RPT)])
    plsc.subcore_barrier()

    def fill(j, _):
        a = ew_f[pl.ds(j * 16, 16)]
        rowi = j * 16 + iota
        plsc.store_scatter(src_v, [rowi, z_i], a)
        return 0

    HG = NG // 2
    for c in range(NCHUNK):
        estart = wid * EPT + c * CH
        pltpu.sync_copy(ei.at[1, pl.ds(estart, CH)], cidx_f)
        pltpu.sync_copy(ew.at[pl.ds(estart, CH)], ew_f)

        lax.fori_loop(0, HG * 5, fill, 0)
        h1 = [pltpu.async_copy(src_v.at[pl.ds(gi * G, G)],
                               deg_sh.at[cidx_f.at[pl.ds(gi * G, G)]],
                               sem_sc, add=True)
              for gi in range(HG)]
        lax.fori_loop(HG * 5, NG * 5, fill, 0)
        h2 = [pltpu.async_copy(src_v.at[pl.ds(gi * G, G)],
                               deg_sh.at[cidx_f.at[pl.ds(gi * G, G)]],
                               sem_sc, add=True)
              for gi in range(HG, NG)]
        for h in h1 + h2:
            h.wait()

    plsc.subcore_barrier()
    tstart = jnp.minimum(sid * TRW, NR - TRW)
    pltpu.sync_copy(deg_sh.at[pl.ds(tstart * 16, TRW * 16)], dstage_v)
    for k in range(TRW):
        pk_v[k, pl.ds(0, 16)] = plsc.load_gather(dstage_v, [k * 16 + iota, z_i])
    pltpu.sync_copy(pk_v, out.at[cid, pl.ds(tstart, TRW)])


def _sc_deg(ei, ew):
    mesh = plsc.VectorSubcoreMesh(core_axis_name="c", subcore_axis_name="s")
    f = functools.partial(
        pl.kernel,
        out_type=jax.ShapeDtypeStruct((2, NR, 16), jnp.float32),
        mesh=mesh,
        compiler_params=pltpu.CompilerParams(use_tc_tiling_on_sc=False, needs_layout_passes=False),
        scratch_types=[
            pltpu.VMEM_SHARED((N, 16), jnp.float32),
            pltpu.VMEM((CH, 16), jnp.float32),
            pltpu.VMEM((CH,), jnp.int32),
            pltpu.VMEM((CH,), jnp.float32),
            pltpu.VMEM((TRW * 16, 16), jnp.float32),
            pltpu.VMEM((TRW, 16), jnp.float32),
            pltpu.SemaphoreType.DMA,
        ],
    )(_sc_deg_body)
    return f(ei, ew)


def _fast_rsqrt(x):
    i = lax.bitcast_convert_type(x, jnp.int32)
    i = 0x5F3759DF - (i >> 1)
    y = lax.bitcast_convert_type(i, jnp.float32)
    for _ in range(3):
        y = y * (1.5 - 0.5 * x * y * y)
    return y


NPT = 640


def _sc_agg_body(ei, ew, hw_hbm, d0_hbm, d1_hbm, out,
                 acc_sh, g_sh, rows_v, ridx_f, cidx_f, ew_f,
                 d0_v, d1_v, dt_v, sem_ga, sem_sc):
    cid = lax.axis_index("c")
    sid = lax.axis_index("s")
    wid = sid * 2 + cid

    z = jnp.zeros((16,), jnp.float32)

    tstart = jnp.minimum(sid * TRW, NR - TRW)
    nstart = tstart * 16
    pltpu.sync_copy(d0_hbm.at[pl.ds(tstart, TRW)], d0_v)
    pltpu.sync_copy(d1_hbm.at[pl.ds(tstart, TRW)], d1_v)
    for k in range(TRW):
        a = d0_v[k, pl.ds(0, 16)]
        b = d1_v[k, pl.ds(0, 16)]
        dt_v[k, pl.ds(0, 16)] = _fast_rsqrt(1.0 + a + b)
    pltpu.sync_copy(hw_hbm.at[pl.ds(nstart, NPT)], rows_v.at[pl.ds(0, NPT)])

    def gscale(k, _):
        dvec = dt_v[k, pl.ds(0, 16)]
        base = k * 16
        for u in range(16):
            s = dvec[u]
            rows_v[base + u, pl.ds(0, 16)] = rows_v[base + u, pl.ds(0, 16)] * s
            rows_v[base + u, pl.ds(16, 16)] = rows_v[base + u, pl.ds(16, 16)] * s
        return 0

    lax.fori_loop(0, TRW, gscale, 0)
    pltpu.sync_copy(rows_v.at[pl.ds(0, NPT)], g_sh.at[pl.ds(nstart, NPT)])

    def zloop(i, _):
        rows_v[i, pl.ds(0, 16)] = z
        rows_v[i, pl.ds(16, 16)] = z
        return 0

    lax.fori_loop(0, CH, zloop, 0)
    pltpu.sync_copy(rows_v.at[pl.ds(0, NPT)], acc_sh.at[pl.ds(nstart, NPT)])
    plsc.subcore_barrier()

    SEGS = [(0, 6), (6, 12), (12, 18), (18, NG)]

    def scale(j, _):
        a = ew_f[pl.ds(j * 16, 16)]
        base = j * 16
        for u in range(16):
            s = a[u]
            rows_v[base + u, pl.ds(0, 16)] = rows_v[base + u, pl.ds(0, 16)] * s
            rows_v[base + u, pl.ds(16, 16)] = rows_v[base + u, pl.ds(16, 16)] * s
        return 0

    for c in range(NCHUNK):
        estart = wid * EPT + c * CH
        pltpu.sync_copy(ei.at[0, pl.ds(estart, CH)], ridx_f)
        pltpu.sync_copy(ei.at[1, pl.ds(estart, CH)], cidx_f)
        pltpu.sync_copy(ew.at[pl.ds(estart, CH)], ew_f)

        ghs = [[pltpu.async_copy(g_sh.at[ridx_f.at[pl.ds(gi * G, G)]],
                                 rows_v.at[pl.ds(gi * G, G)], sem_ga)
                for gi in range(lo, hi)]
               for lo, hi in SEGS]
        shs = []
        for (lo, hi), gh in zip(SEGS, ghs):
            for h in gh:
                h.wait()
            lax.fori_loop(lo * 5, hi * 5, scale, 0)
            shs += [pltpu.async_copy(rows_v.at[pl.ds(gi * G, G)],
                                     acc_sh.at[cidx_f.at[pl.ds(gi * G, G)]],
                                     sem_sc, add=True)
                    for gi in range(lo, hi)]
        for h in shs:
            h.wait()

    plsc.subcore_barrier()
    pltpu.sync_copy(acc_sh.at[pl.ds(nstart, NPT)],
                    out.at[cid, pl.ds(nstart, NPT)])


def _sc_agg(ei, ew, hw, d0, d1):
    mesh = plsc.VectorSubcoreMesh(core_axis_name="c", subcore_axis_name="s")
    f = functools.partial(
        pl.kernel,
        out_type=jax.ShapeDtypeStruct((2, N, H2), jnp.float32),
        mesh=mesh,
        compiler_params=pltpu.CompilerParams(use_tc_tiling_on_sc=False, needs_layout_passes=False),
        scratch_types=[
            pltpu.VMEM_SHARED((N, H2), jnp.float32),
            pltpu.VMEM_SHARED((N, H2), jnp.float32),
            pltpu.VMEM((CH, H2), jnp.float32),
            pltpu.VMEM((CH,), jnp.int32),
            pltpu.VMEM((CH,), jnp.int32),
            pltpu.VMEM((CH,), jnp.float32),
            pltpu.VMEM((TRW, 16), jnp.float32),
            pltpu.VMEM((TRW, 16), jnp.float32),
            pltpu.VMEM((TRW, 16), jnp.float32),
            pltpu.SemaphoreType.DMA,
            pltpu.SemaphoreType.DMA,
        ],
    )(_sc_agg_body)
    return f(ei, ew, hw, d0, d1)


def _tc_out_body(h_ref, hw_ref, degp_ref, accp_ref, bc_ref, g2_ref, bb2_ref,
                 w2_ref, b2_ref, o_ref):
    deg = 1.0 + degp_ref[0] + degp_ref[1]
    dinv = lax.rsqrt(deg)
    acc = accp_ref[0] + accp_ref[1] + dinv * hw_ref[...]
    conv = dinv * acc + bc_ref[...][None, :]
    t = jnp.where(conv >= 0, conv, 0.01 * conv)
    h2 = t * (g2_ref[...] * _BN_INV)[None, :] + bb2_ref[...][None, :]
    logits = (jnp.dot(h_ref[...], w2_ref[0:H, :], preferred_element_type=jnp.float32)
              + jnp.dot(h2, w2_ref[H:H + H2, :], preferred_element_type=jnp.float32)
              + b2_ref[...][None, :])
    m = jnp.max(logits, axis=1, keepdims=True)
    zc = logits - m
    lse = jnp.log(jnp.sum(jnp.exp(zc), axis=1, keepdims=True))
    o_ref[...] = zc - lse


def _tc_out(h, hw, degp, accp, bc, bn2_g, bn2_b, W2, b2):
    return pl.pallas_call(
        _tc_out_body,
        grid=(GRID,),
        in_specs=[
            pl.BlockSpec((RB, H), lambda i: (i, 0)),
            pl.BlockSpec((RB, H2), lambda i: (i, 0)),
            pl.BlockSpec((2, RB, 1), lambda i: (0, i, 0)),
            pl.BlockSpec((2, RB, H2), lambda i: (0, i, 0)),
            pl.BlockSpec((H2,), lambda i: (0,)),
            pl.BlockSpec((H2,), lambda i: (0,)),
            pl.BlockSpec((H2,), lambda i: (0,)),
            pl.BlockSpec((H + H2, C), lambda i: (0, 0)),
            pl.BlockSpec((C,), lambda i: (0,)),
        ],
        out_specs=pl.BlockSpec((RB, C), lambda i: (i, 0)),
        out_shape=jax.ShapeDtypeStruct((N, C), jnp.float32),
    )(h, hw, degp, accp, bc, bn2_g, bn2_b, W2, b2)


def kernel(x, edge_index, edge_weight, W1, b1, Wc, bc, W2, b2,
           bn1_g, bn1_b, bn2_g, bn2_b):
    degp = _sc_deg(edge_index, edge_weight)
    h, hw = _tc_fwd(x, W1, b1, Wc, bn1_g, bn1_b)
    accp = _sc_agg(edge_index, edge_weight, hw, degp[0], degp[1])
    degf = degp.reshape(2, N, 1)
    return _tc_out(h, hw, degf, accp, bc, bn2_g, bn2_b, W2, b2)


# speedup vs baseline: 1.0321x; 1.0321x over previous
import functools

import jax
import jax.numpy as jnp
from jax import lax
from jax.experimental import pallas as pl
from jax.experimental.pallas import tpu as pltpu
from jax.experimental.pallas import tpu_sc as plsc

N = 10000
E = 320000
F_IN = 128
H = 64
H2 = 32
C = 40

_BN_INV = (1.0 + 1e-5) ** -0.5

NW = 32
EPT = E // NW
G = 80
CH = 2000
NG = CH // G
NCHUNK = EPT // CH
RPT = N // 16
RB = 2000
GRID = N // RB


def _tc_fwd_body(x_ref, w1_ref, b1_ref, wc_ref, g1_ref, bb1_ref, h_ref, hw_ref):
    h = jnp.dot(x_ref[...], w1_ref[...], preferred_element_type=jnp.float32)
    h = h + b1_ref[...][None, :]
    h = jnp.where(h >= 0, h, 0.01 * h)
    h = h * (g1_ref[...] * _BN_INV)[None, :] + bb1_ref[...][None, :]
    h_ref[...] = h
    hw_ref[...] = jnp.dot(h, wc_ref[...], preferred_element_type=jnp.float32)


def _tc_fwd(x, W1, b1, Wc, bn1_g, bn1_b):
    return pl.pallas_call(
        _tc_fwd_body,
        grid=(GRID,),
        in_specs=[
            pl.BlockSpec((RB, F_IN), lambda i: (i, 0)),
            pl.BlockSpec((F_IN, H), lambda i: (0, 0)),
            pl.BlockSpec((H,), lambda i: (0,)),
            pl.BlockSpec((H, H2), lambda i: (0, 0)),
            pl.BlockSpec((H,), lambda i: (0,)),
            pl.BlockSpec((H,), lambda i: (0,)),
        ],
        out_specs=[
            pl.BlockSpec((RB, H), lambda i: (i, 0)),
            pl.BlockSpec((RB, H2), lambda i: (i, 0)),
        ],
        out_shape=[
            jax.ShapeDtypeStruct((N, H), jnp.float32),
            jax.ShapeDtypeStruct((N, H2), jnp.float32),
        ],
    )(x, W1, b1, Wc, bn1_g, bn1_b)


NR = N // 16
TRW = 40


def _sc_deg_body(ei, ew, out, deg_sh, src_v, cidx_f, ew_f, dstage_v, pk_v,
                 sem_sc):
    cid = lax.axis_index("c")
    sid = lax.axis_index("s")
    wid = sid * 2 + cid

    z = jnp.zeros((16,), jnp.float32)
    z_i = jnp.zeros((16,), jnp.int32)
    iota = lax.iota(jnp.int32, 16)

    def zloop(i, _):
        src_v[i, pl.ds(0, 16)] = z
        return 0

    lax.fori_loop(0, CH, zloop, 0)
    pltpu.sync_copy(src_v.at[pl.ds(0, RPT)],
                    deg_sh.at[pl.ds(sid * RPT, RPT)])
    plsc.subcore_barrier()

    def fill(j, _):
        a = ew_f[pl.ds(j * 16, 16)]
        rowi = j * 16 + iota
        plsc.store_scatter(src_v, [rowi, z_i], a)
        return 0

    HG = NG // 2
    for c in range(NCHUNK):
        estart = wid * EPT + c * CH
        pltpu.sync_copy(ei.at[1, pl.ds(estart, CH)], cidx_f)
        pltpu.sync_copy(ew.at[pl.ds(estart, CH)], ew_f)

        lax.fori_loop(0, HG * 5, fill, 0)
        h1 = [pltpu.async_copy(src_v.at[pl.ds(gi * G, G)],
                               deg_sh.at[cidx_f.at[pl.ds(gi * G, G)]],
                               sem_sc, add=True)
              for gi in range(HG)]
        lax.fori_loop(HG * 5, NG * 5, fill, 0)
        h2 = [pltpu.async_copy(src_v.at[pl.ds(gi * G, G)],
                               deg_sh.at[cidx_f.at[pl.ds(gi * G, G)]],
                               sem_sc, add=True)
              for gi in range(HG, NG)]
        for h in h1 + h2:
            h.wait()

    plsc.subcore_barrier()
    tstart = jnp.minimum(sid * TRW, NR - TRW)
    pltpu.sync_copy(deg_sh.at[pl.ds(tstart * 16, TRW * 16)], dstage_v)
    for k in range(TRW):
        pk_v[k, pl.ds(0, 16)] = plsc.load_gather(dstage_v, [k * 16 + iota, z_i])
    pltpu.sync_copy(pk_v, out.at[cid, pl.ds(tstart, TRW)])


def _sc_deg(ei, ew):
    mesh = plsc.VectorSubcoreMesh(core_axis_name="c", subcore_axis_name="s")
    f = functools.partial(
        pl.kernel,
        out_type=jax.ShapeDtypeStruct((2, NR, 16), jnp.float32),
        mesh=mesh,
        compiler_params=pltpu.CompilerParams(use_tc_tiling_on_sc=False, needs_layout_passes=False),
        scratch_types=[
            pltpu.VMEM_SHARED((N, 16), jnp.float32),
            pltpu.VMEM((CH, 16), jnp.float32),
            pltpu.VMEM((CH,), jnp.int32),
            pltpu.VMEM((CH,), jnp.float32),
            pltpu.VMEM((TRW * 16, 16), jnp.float32),
            pltpu.VMEM((TRW, 16), jnp.float32),
            pltpu.SemaphoreType.DMA,
        ],
    )(_sc_deg_body)
    return f(ei, ew)


def _fast_rsqrt(x):
    i = lax.bitcast_convert_type(x, jnp.int32)
    i = 0x5F3759DF - (i >> 1)
    y = lax.bitcast_convert_type(i, jnp.float32)
    for _ in range(3):
        y = y * (1.5 - 0.5 * x * y * y)
    return y


NPT = 640


def _sc_agg_body(ei, ew, hw_hbm, d0_hbm, d1_hbm, out,
                 acc_sh, g_sh, rows_v, ridx_f, cidx_f, ew_f,
                 d0_v, d1_v, dt_v, sem_ga, sem_sc):
    cid = lax.axis_index("c")
    sid = lax.axis_index("s")
    wid = sid * 2 + cid

    z = jnp.zeros((16,), jnp.float32)

    tstart = jnp.minimum(sid * TRW, NR - TRW)
    nstart = tstart * 16
    pltpu.sync_copy(d0_hbm.at[pl.ds(tstart, TRW)], d0_v)
    pltpu.sync_copy(d1_hbm.at[pl.ds(tstart, TRW)], d1_v)
    for k in range(TRW):
        a = d0_v[k, pl.ds(0, 16)]
        b = d1_v[k, pl.ds(0, 16)]
        dt_v[k, pl.ds(0, 16)] = _fast_rsqrt(1.0 + a + b)
    pltpu.sync_copy(hw_hbm.at[pl.ds(nstart, NPT)], rows_v.at[pl.ds(0, NPT)])

    def gscale(k, _):
        dvec = dt_v[k, pl.ds(0, 16)]
        base = k * 16
        for u in range(16):
            s = dvec[u]
            rows_v[base + u, pl.ds(0, 16)] = rows_v[base + u, pl.ds(0, 16)] * s
            rows_v[base + u, pl.ds(16, 16)] = rows_v[base + u, pl.ds(16, 16)] * s
        return 0

    lax.fori_loop(0, TRW, gscale, 0)
    pltpu.sync_copy(rows_v.at[pl.ds(0, NPT)], g_sh.at[pl.ds(nstart, NPT)])

    def zloop(i, _):
        rows_v[i, pl.ds(0, 16)] = z
        rows_v[i, pl.ds(16, 16)] = z
        return 0

    lax.fori_loop(0, CH, zloop, 0)
    pltpu.sync_copy(rows_v.at[pl.ds(0, NPT)], acc_sh.at[pl.ds(nstart, NPT)])
    plsc.subcore_barrier()

    SEGS = [(0, 12), (12, NG)]

    def scale(j, _):
        a = ew_f[pl.ds(j * 16, 16)]
        base = j * 16
        for u in range(16):
            s = a[u]
            rows_v[base + u, pl.ds(0, 16)] = rows_v[base + u, pl.ds(0, 16)] * s
            rows_v[base + u, pl.ds(16, 16)] = rows_v[base + u, pl.ds(16, 16)] * s
        return 0

    for c in range(NCHUNK):
        estart = wid * EPT + c * CH
        pltpu.sync_copy(ei.at[0, pl.ds(estart, CH)], ridx_f)
        pltpu.sync_copy(ei.at[1, pl.ds(estart, CH)], cidx_f)
        pltpu.sync_copy(ew.at[pl.ds(estart, CH)], ew_f)

        ghs = [[pltpu.async_copy(g_sh.at[ridx_f.at[pl.ds(gi * G, G)]],
                                 rows_v.at[pl.ds(gi * G, G)], sem_ga)
                for gi in range(lo, hi)]
               for lo, hi in SEGS]
        shs = []
        for (lo, hi), gh in zip(SEGS, ghs):
            for h in gh:
                h.wait()
            lax.fori_loop(lo * 5, hi * 5, scale, 0)
            shs += [pltpu.async_copy(rows_v.at[pl.ds(gi * G, G)],
                                     acc_sh.at[cidx_f.at[pl.ds(gi * G, G)]],
                                     sem_sc, add=True)
                    for gi in range(lo, hi)]
        for h in shs:
            h.wait()

    plsc.subcore_barrier()
    pltpu.sync_copy(acc_sh.at[pl.ds(nstart, NPT)],
                    out.at[cid, pl.ds(nstart, NPT)])


def _sc_agg(ei, ew, hw, d0, d1):
    mesh = plsc.VectorSubcoreMesh(core_axis_name="c", subcore_axis_name="s")
    f = functools.partial(
        pl.kernel,
        out_type=jax.ShapeDtypeStruct((2, N, H2), jnp.float32),
        mesh=mesh,
        compiler_params=pltpu.CompilerParams(use_tc_tiling_on_sc=False, needs_layout_passes=False),
        scratch_types=[
            pltpu.VMEM_SHARED((N, H2), jnp.float32),
            pltpu.VMEM_SHARED((N, H2), jnp.float32),
            pltpu.VMEM((CH, H2), jnp.float32),
            pltpu.VMEM((CH,), jnp.int32),
            pltpu.VMEM((CH,), jnp.int32),
            pltpu.VMEM((CH,), jnp.float32),
            pltpu.VMEM((TRW, 16), jnp.float32),
            pltpu.VMEM((TRW, 16), jnp.float32),
            pltpu.VMEM((TRW, 16), jnp.float32),
            pltpu.SemaphoreType.DMA,
            pltpu.SemaphoreType.DMA,
        ],
    )(_sc_agg_body)
    return f(ei, ew, hw, d0, d1)


def _tc_out_body(h_ref, hw_ref, degp_ref, accp_ref, bc_ref, g2_ref, bb2_ref,
                 w2_ref, b2_ref, o_ref):
    deg = 1.0 + degp_ref[0] + degp_ref[1]
    dinv = lax.rsqrt(deg)
    acc = accp_ref[0] + accp_ref[1] + dinv * hw_ref[...]
    conv = dinv * acc + bc_ref[...][None, :]
    t = jnp.where(conv >= 0, conv, 0.01 * conv)
    h2 = t * (g2_ref[...] * _BN_INV)[None, :] + bb2_ref[...][None, :]
    logits = (jnp.dot(h_ref[...], w2_ref[0:H, :], preferred_element_type=jnp.float32)
              + jnp.dot(h2, w2_ref[H:H + H2, :], preferred_element_type=jnp.float32)
              + b2_ref[...][None, :])
    m = jnp.max(logits, axis=1, keepdims=True)
    zc = logits - m
    lse = jnp.log(jnp.sum(jnp.exp(zc), axis=1, keepdims=True))
    o_ref[...] = zc - lse


def _tc_out(h, hw, degp, accp, bc, bn2_g, bn2_b, W2, b2):
    return pl.pallas_call(
        _tc_out_body,
        grid=(GRID,),
        in_specs=[
            pl.BlockSpec((RB, H), lambda i: (i, 0)),
            pl.BlockSpec((RB, H2), lambda i: (i, 0)),
            pl.BlockSpec((2, RB, 1), lambda i: (0, i, 0)),
            pl.BlockSpec((2, RB, H2), lambda i: (0, i, 0)),
            pl.BlockSpec((H2,), lambda i: (0,)),
            pl.BlockSpec((H2,), lambda i: (0,)),
            pl.BlockSpec((H2,), lambda i: (0,)),
            pl.BlockSpec((H + H2, C), lambda i: (0, 0)),
            pl.BlockSpec((C,), lambda i: (0,)),
        ],
        out_specs=pl.BlockSpec((RB, C), lambda i: (i, 0)),
        out_shape=jax.ShapeDtypeStruct((N, C), jnp.float32),
    )(h, hw, degp, accp, bc, bn2_g, bn2_b, W2, b2)


def kernel(x, edge_index, edge_weight, W1, b1, Wc, bc, W2, b2,
           bn1_g, bn1_b, bn2_g, bn2_b):
    degp = _sc_deg(edge_index, edge_weight)
    h, hw = _tc_fwd(x, W1, b1, Wc, bn1_g, bn1_b)
    accp = _sc_agg(edge_index, edge_weight, hw, degp[0], degp[1])
    degf = degp.reshape(2, N, 1)
    return _tc_out(h, hw, degf, accp, bc, bn2_g, bn2_b, W2, b2)
